# initial kernel scaffold (unmeasured)
import jax
import jax.numpy as jnp
from jax import lax
from jax.experimental import pallas as pl
from jax.experimental.pallas import tpu as pltpu

N_DEV = 4
M = 8192
D = 2048
CH = M // N_DEV


def kernel(partial, resid, gamma):
    partial = partial.reshape(M, D)
    gamma = gamma.reshape(1, D)

    def body(p_ref, r_ref, g_ref, out_ref,
             acc, stage, rs_comm, ag_comm,
             rs_send, rs_recv, ag_send, ag_recv,
             load_sem, store_sem):
        my = lax.axis_index("i")
        left = jnp.mod(my - 1, N_DEV)
        right = jnp.mod(my + 1, N_DEV)

        barrier = pltpu.get_barrier_semaphore()
        for nbr in (left, right):
            pl.semaphore_signal(barrier, inc=1, device_id=(nbr,),
                                device_id_type=pl.DeviceIdType.MESH)
        pl.semaphore_wait(barrier, 2)

        def row_block(ref, c):
            return ref.at[pl.ds(c * CH, CH), :]

        cp = pltpu.make_async_copy(row_block(p_ref, my), acc, load_sem)
        cp.start()
        cp.wait()

        for s in range(N_DEV - 1):
            slot = s % 2
            rdma = pltpu.make_async_remote_copy(
                src_ref=acc, dst_ref=rs_comm.at[slot],
                send_sem=rs_send.at[slot], recv_sem=rs_recv.at[slot],
                device_id=(right,), device_id_type=pl.DeviceIdType.MESH)
            rdma.start()
            c_next = jnp.mod(my - s - 1, N_DEV)
            cp = pltpu.make_async_copy(row_block(p_ref, c_next), stage,
                                       load_sem)
            cp.start()
            cp.wait()
            rdma.wait()
            acc[...] = rs_comm[slot] + stage[...]

        own = right
        cp = pltpu.make_async_copy(row_block(r_ref, own), stage, load_sem)
        cp.start()
        cp.wait()

        y = acc[...] + stage[...]
        ms = jnp.mean(y * y, axis=-1, keepdims=True)
        acc[...] = y * lax.rsqrt(ms + 1e-6) * g_ref[...]

        st = pltpu.make_async_copy(acc, row_block(out_ref, own), store_sem)
        st.start()
        st.wait()

        for h in range(N_DEV - 1):
            slot = h % 2
            src = acc if h == 0 else ag_comm.at[(h - 1) % 2]
            rdma = pltpu.make_async_remote_copy(
                src_ref=src, dst_ref=ag_comm.at[slot],
                send_sem=ag_send.at[slot], recv_sem=ag_recv.at[slot],
                device_id=(right,), device_id_type=pl.DeviceIdType.MESH)
            rdma.start()
            rdma.wait()
            origin = jnp.mod(my - h, N_DEV)
            st = pltpu.make_async_copy(ag_comm.at[slot],
                                       row_block(out_ref, origin), store_sem)
            st.start()
            st.wait()

    return pl.pallas_call(
        body,
        out_shape=jax.ShapeDtypeStruct((M, D), jnp.float32),
        in_specs=[
            pl.BlockSpec(memory_space=pltpu.ANY),
            pl.BlockSpec(memory_space=pltpu.ANY),
            pl.BlockSpec(memory_space=pltpu.VMEM),
        ],
        out_specs=pl.BlockSpec(memory_space=pltpu.ANY),
        scratch_shapes=[
            pltpu.VMEM((CH, D), jnp.float32),
            pltpu.VMEM((CH, D), jnp.float32),
            pltpu.VMEM((2, CH, D), jnp.float32),
            pltpu.VMEM((2, CH, D), jnp.float32),
            pltpu.SemaphoreType.DMA((2,)),
            pltpu.SemaphoreType.DMA((2,)),
            pltpu.SemaphoreType.DMA((2,)),
            pltpu.SemaphoreType.DMA((2,)),
            pltpu.SemaphoreType.DMA,
            pltpu.SemaphoreType.DMA,
        ],
        compiler_params=pltpu.CompilerParams(
            collective_id=0,
            vmem_limit_bytes=128 * 1024 * 1024,
        ),
    )(partial, resid, gamma)


# baseline (device time: 1215515 ns/iter reference)
import jax
import jax.numpy as jnp
from jax import lax
from jax.experimental import pallas as pl
from jax.experimental.pallas import tpu as pltpu

N_DEV = 4
M = 8192
D = 2048
CH = M // N_DEV
TR = 512
NT = CH // TR


def kernel(partial, resid, gamma):
    partial = partial.reshape(M, D)
    gamma = gamma.reshape(1, D)

    def body(p_ref, r_ref, g_ref, out_ref, rbuf,
             va, vb, vc,
             rs_send, rs_recv, ag_send, ag_recv,
             sem_a, sem_b, sem_c, sem_out):
        my = lax.axis_index("i")
        left = jnp.mod(my - 1, N_DEV)
        right = jnp.mod(my + 1, N_DEV)

        barrier = pltpu.get_barrier_semaphore()
        for nbr in (left, right):
            pl.semaphore_signal(barrier, inc=1, device_id=(nbr,),
                                device_id_type=pl.DeviceIdType.MESH)
        pl.semaphore_wait(barrier, 2)

        def rows(ref, c, t=None):
            if t is None:
                return ref.at[pl.ds(c * CH, CH), :]
            return ref.at[pl.ds(c * CH + t * TR, TR), :]

        for s in range(N_DEV - 1):
            src = rows(p_ref, my) if s == 0 else rbuf.at[s - 1]
            rdma = pltpu.make_async_remote_copy(
                src_ref=src, dst_ref=rbuf.at[s],
                send_sem=rs_send.at[s], recv_sem=rs_recv.at[s],
                device_id=(right,), device_id_type=pl.DeviceIdType.MESH)
            rdma.start()
            rdma.wait()
            c_in = jnp.mod(my - s - 1, N_DEV)
            if s < N_DEV - 2:
                for t in range(NT):
                    cpa = pltpu.make_async_copy(
                        rbuf.at[s, pl.ds(t * TR, TR), :], va, sem_a)
                    cpb = pltpu.make_async_copy(
                        rows(p_ref, c_in, t), vb, sem_b)
                    cpa.start()
                    cpb.start()
                    cpa.wait()
                    cpb.wait()
                    va[...] = va[...] + vb[...]
                    st = pltpu.make_async_copy(
                        va, rbuf.at[s, pl.ds(t * TR, TR), :], sem_a)
                    st.start()
                    st.wait()

        own = right
        for t in range(NT):
            cpa = pltpu.make_async_copy(
                rbuf.at[N_DEV - 2, pl.ds(t * TR, TR), :], va, sem_a)
            cpb = pltpu.make_async_copy(rows(p_ref, own, t), vb, sem_b)
            cpc = pltpu.make_async_copy(rows(r_ref, own, t), vc, sem_c)
            cpa.start()
            cpb.start()
            cpc.start()
            cpa.wait()
            cpb.wait()
            cpc.wait()
            y = va[...] + vb[...] + vc[...]
            ms = jnp.mean(y * y, axis=-1, keepdims=True)
            va[...] = y * lax.rsqrt(ms + 1e-6) * g_ref[...]
            st = pltpu.make_async_copy(va, rows(out_ref, own, t), sem_out)
            st.start()
            st.wait()

        for h in range(N_DEV - 1):
            c_h = jnp.mod(my + 1 - h, N_DEV)
            rdma = pltpu.make_async_remote_copy(
                src_ref=rows(out_ref, c_h), dst_ref=rows(out_ref, c_h),
                send_sem=ag_send.at[h], recv_sem=ag_recv.at[h],
                device_id=(right,), device_id_type=pl.DeviceIdType.MESH)
            rdma.start()
            rdma.wait()

    out, _ = pl.pallas_call(
        body,
        out_shape=(
            jax.ShapeDtypeStruct((M, D), jnp.float32),
            jax.ShapeDtypeStruct((N_DEV - 1, CH, D), jnp.float32),
        ),
        in_specs=[
            pl.BlockSpec(memory_space=pl.ANY),
            pl.BlockSpec(memory_space=pl.ANY),
            pl.BlockSpec(memory_space=pltpu.MemorySpace.VMEM),
        ],
        out_specs=(
            pl.BlockSpec(memory_space=pl.ANY),
            pl.BlockSpec(memory_space=pl.ANY),
        ),
        scratch_shapes=[
            pltpu.VMEM((TR, D), jnp.float32),
            pltpu.VMEM((TR, D), jnp.float32),
            pltpu.VMEM((TR, D), jnp.float32),
            pltpu.SemaphoreType.DMA((N_DEV - 1,)),
            pltpu.SemaphoreType.DMA((N_DEV - 1,)),
            pltpu.SemaphoreType.DMA((N_DEV - 1,)),
            pltpu.SemaphoreType.DMA((N_DEV - 1,)),
            pltpu.SemaphoreType.DMA,
            pltpu.SemaphoreType.DMA,
            pltpu.SemaphoreType.DMA,
            pltpu.SemaphoreType.DMA,
        ],
        compiler_params=pltpu.CompilerParams(
            collective_id=0,
            vmem_limit_bytes=64 * 1024 * 1024,
        ),
    )(partial, resid, gamma)
    return out


# device time: 676780 ns/iter; 1.7960x vs baseline; 1.7960x over previous
import jax
import jax.numpy as jnp
from jax import lax
from jax.experimental import pallas as pl
from jax.experimental.pallas import tpu as pltpu

N_DEV = 4
M = 8192
D = 2048
W = D // 2
CH = M // N_DEV
TR = 512
NT = CH // TR


def kernel(partial, resid, gamma):
    partial = partial.reshape(M, D)
    gamma = gamma.reshape(1, D)

    def body(p_ref, r_ref, g_ref, out_ref, rbuf,
             va, vb, vc,
             rs_send, rs_recv, ag_send, ag_recv,
             sem_a, sem_b, sem_b2, sem_c, sem_out):
        my = lax.axis_index("i")
        left = jnp.mod(my - 1, N_DEV)
        right = jnp.mod(my + 1, N_DEV)

        barrier = pltpu.get_barrier_semaphore()
        for nbr in (left, right):
            pl.semaphore_signal(barrier, inc=1, device_id=(nbr,),
                                device_id_type=pl.DeviceIdType.MESH)
        pl.semaphore_wait(barrier, 2)

        def rows(ref, c, t=None):
            if t is None:
                return ref.at[pl.ds(c * CH, CH), :]
            return ref.at[pl.ds(c * CH + t * TR, TR), :]

        def rows_a(ref, c):
            return ref.at[pl.ds(c * CH, CH), 0:W]

        def rows_b(ref, c):
            return ref.at[pl.ds(c * CH, CH), W:D]

        for s in range(N_DEV - 1):
            src_a = rows_a(p_ref, my) if s == 0 else rbuf.at[s - 1, :, 0:W]
            rdma_a = pltpu.make_async_remote_copy(
                src_ref=src_a, dst_ref=rbuf.at[s, :, 0:W],
                send_sem=rs_send.at[s], recv_sem=rs_recv.at[s],
                device_id=(right,), device_id_type=pl.DeviceIdType.MESH)
            c_b0 = jnp.mod(my + 2, N_DEV)
            src_b = rows_b(p_ref, c_b0) if s == 0 else rbuf.at[s - 1, :, W:D]
            rdma_b = pltpu.make_async_remote_copy(
                src_ref=src_b, dst_ref=rbuf.at[s, :, W:D],
                send_sem=rs_send.at[3 + s], recv_sem=rs_recv.at[3 + s],
                device_id=(left,), device_id_type=pl.DeviceIdType.MESH)
            rdma_a.start()
            rdma_b.start()
            rdma_a.wait()
            rdma_b.wait()
            if s < N_DEV - 2:
                c_a = jnp.mod(my - s - 1, N_DEV)
                c_b = jnp.mod(my + s + 3, N_DEV)
                for t in range(NT):
                    cpa = pltpu.make_async_copy(
                        rbuf.at[s, pl.ds(t * TR, TR), :], va, sem_a)
                    cpb = pltpu.make_async_copy(
                        p_ref.at[pl.ds(c_a * CH + t * TR, TR), 0:W],
                        vb.at[:, 0:W], sem_b)
                    cpb2 = pltpu.make_async_copy(
                        p_ref.at[pl.ds(c_b * CH + t * TR, TR), W:D],
                        vb.at[:, W:D], sem_b2)
                    cpa.start()
                    cpb.start()
                    cpb2.start()
                    cpa.wait()
                    cpb.wait()
                    cpb2.wait()
                    va[...] = va[...] + vb[...]
                    st = pltpu.make_async_copy(
                        va, rbuf.at[s, pl.ds(t * TR, TR), :], sem_a)
                    st.start()
                    st.wait()

        own = right
        for t in range(NT):
            cpa = pltpu.make_async_copy(
                rbuf.at[N_DEV - 2, pl.ds(t * TR, TR), :], va, sem_a)
            cpb = pltpu.make_async_copy(rows(p_ref, own, t), vb, sem_b)
            cpc = pltpu.make_async_copy(rows(r_ref, own, t), vc, sem_c)
            cpa.start()
            cpb.start()
            cpc.start()
            cpa.wait()
            cpb.wait()
            cpc.wait()
            y = va[...] + vb[...] + vc[...]
            ms = jnp.mean(y * y, axis=-1, keepdims=True)
            va[...] = y * lax.rsqrt(ms + 1e-6) * g_ref[...]
            st = pltpu.make_async_copy(va, rows(out_ref, own, t), sem_out)
            st.start()
            st.wait()

        for h in range(N_DEV - 1):
            c_a = jnp.mod(my + 1 - h, N_DEV)
            rdma_a = pltpu.make_async_remote_copy(
                src_ref=rows_a(out_ref, c_a), dst_ref=rows_a(out_ref, c_a),
                send_sem=ag_send.at[h], recv_sem=ag_recv.at[h],
                device_id=(right,), device_id_type=pl.DeviceIdType.MESH)
            c_b = jnp.mod(my + 1 + h, N_DEV)
            rdma_b = pltpu.make_async_remote_copy(
                src_ref=rows_b(out_ref, c_b), dst_ref=rows_b(out_ref, c_b),
                send_sem=ag_send.at[3 + h], recv_sem=ag_recv.at[3 + h],
                device_id=(left,), device_id_type=pl.DeviceIdType.MESH)
            rdma_a.start()
            rdma_b.start()
            rdma_a.wait()
            rdma_b.wait()

    out, _ = pl.pallas_call(
        body,
        out_shape=(
            jax.ShapeDtypeStruct((M, D), jnp.float32),
            jax.ShapeDtypeStruct((N_DEV - 1, CH, D), jnp.float32),
        ),
        in_specs=[
            pl.BlockSpec(memory_space=pl.ANY),
            pl.BlockSpec(memory_space=pl.ANY),
            pl.BlockSpec(memory_space=pltpu.MemorySpace.VMEM),
        ],
        out_specs=(
            pl.BlockSpec(memory_space=pl.ANY),
            pl.BlockSpec(memory_space=pl.ANY),
        ),
        scratch_shapes=[
            pltpu.VMEM((TR, D), jnp.float32),
            pltpu.VMEM((TR, D), jnp.float32),
            pltpu.VMEM((TR, D), jnp.float32),
            pltpu.SemaphoreType.DMA((2 * (N_DEV - 1),)),
            pltpu.SemaphoreType.DMA((2 * (N_DEV - 1),)),
            pltpu.SemaphoreType.DMA((2 * (N_DEV - 1),)),
            pltpu.SemaphoreType.DMA((2 * (N_DEV - 1),)),
            pltpu.SemaphoreType.DMA,
            pltpu.SemaphoreType.DMA,
            pltpu.SemaphoreType.DMA,
            pltpu.SemaphoreType.DMA,
            pltpu.SemaphoreType.DMA,
        ],
        compiler_params=pltpu.CompilerParams(
            collective_id=0,
            vmem_limit_bytes=64 * 1024 * 1024,
        ),
    )(partial, resid, gamma)
    return out


# device time: 590109 ns/iter; 2.0598x vs baseline; 1.1469x over previous
import jax
import jax.numpy as jnp
from jax import lax
from jax.experimental import pallas as pl
from jax.experimental.pallas import tpu as pltpu

N_DEV = 4
N_HOP = N_DEV - 1
M = 8192
D = 2048
W = D // 2
CH = M // N_DEV
TR = 512
SB = CH // TR


def kernel(partial, resid, gamma):
    partial = partial.reshape(M, D)
    gamma = gamma.reshape(1, D)

    def body(p_ref, r_ref, g_ref, out_ref, rbuf,
             va, vb, vc,
             rs_send, rs_recv, ag_send, ag_recv,
             sem_a, sem_b, sem_b2, sem_c, sem_out):
        my = lax.axis_index("i")
        left = jnp.mod(my - 1, N_DEV)
        right = jnp.mod(my + 1, N_DEV)
        own = right

        barrier = pltpu.get_barrier_semaphore()
        for nbr in (left, right):
            pl.semaphore_signal(barrier, inc=1, device_id=(nbr,),
                                device_id_type=pl.DeviceIdType.MESH)
        pl.semaphore_wait(barrier, 2)

        def sub(c, k):
            return pl.ds(c * CH + k * TR, TR)

        def idx(d, s, k):
            return (d * N_HOP + s) * SB + k

        def mk(src, dst, sems, d, s, k, dev):
            ssem, rsem = sems
            return pltpu.make_async_remote_copy(
                src_ref=src, dst_ref=dst,
                send_sem=ssem.at[idx(d, s, k)],
                recv_sem=rsem.at[idx(d, s, k)],
                device_id=(dev,), device_id_type=pl.DeviceIdType.MESH)

        rs_sems = (rs_send, rs_recv)
        ag_sems = (ag_send, ag_recv)
        all_desc = []
        rs_desc = {}
        ag_desc = {}

        def rs_start(s, k):
            if s == 0:
                c_b0 = jnp.mod(my + 2, N_DEV)
                src_a = p_ref.at[sub(my, k), 0:W]
                src_b = p_ref.at[sub(c_b0, k), W:D]
            else:
                src_a = rbuf.at[s - 1, pl.ds(k * TR, TR), 0:W]
                src_b = rbuf.at[s - 1, pl.ds(k * TR, TR), W:D]
            ra = mk(src_a, rbuf.at[s, pl.ds(k * TR, TR), 0:W],
                    rs_sems, 0, s, k, right)
            rb = mk(src_b, rbuf.at[s, pl.ds(k * TR, TR), W:D],
                    rs_sems, 1, s, k, left)
            ra.start()
            rb.start()
            rs_desc[(0, s, k)] = ra
            rs_desc[(1, s, k)] = rb
            all_desc.extend([ra, rb])

        def ag_start(h, k):
            c_a = jnp.mod(my + 1 - h, N_DEV)
            c_b = jnp.mod(my + 1 + h, N_DEV)
            ra = mk(out_ref.at[sub(c_a, k), 0:W],
                    out_ref.at[sub(c_a, k), 0:W], ag_sems, 0, h, k, right)
            rb = mk(out_ref.at[sub(c_b, k), W:D],
                    out_ref.at[sub(c_b, k), W:D], ag_sems, 1, h, k, left)
            ra.start()
            rb.start()
            ag_desc[(0, h, k)] = ra
            ag_desc[(1, h, k)] = rb
            all_desc.extend([ra, rb])

        for k in range(SB):
            rs_start(0, k)

        for s in range(N_HOP):
            for k in range(SB):
                rs_desc[(0, s, k)].wait_recv()
                rs_desc[(1, s, k)].wait_recv()
                if s < N_HOP - 1:
                    c_a = jnp.mod(my - s - 1, N_DEV)
                    c_b = jnp.mod(my + s + 3, N_DEV)
                    cpa = pltpu.make_async_copy(
                        rbuf.at[s, pl.ds(k * TR, TR), :], va, sem_a)
                    cpb = pltpu.make_async_copy(
                        p_ref.at[sub(c_a, k), 0:W], vb.at[:, 0:W], sem_b)
                    cpb2 = pltpu.make_async_copy(
                        p_ref.at[sub(c_b, k), W:D], vb.at[:, W:D], sem_b2)
                    cpa.start()
                    cpb.start()
                    cpb2.start()
                    cpa.wait()
                    cpb.wait()
                    cpb2.wait()
                    va[...] = va[...] + vb[...]
                    st = pltpu.make_async_copy(
                        va, rbuf.at[s, pl.ds(k * TR, TR), :], sem_a)
                    st.start()
                    st.wait()
                    rs_start(s + 1, k)
                else:
                    cpa = pltpu.make_async_copy(
                        rbuf.at[s, pl.ds(k * TR, TR), :], va, sem_a)
                    cpb = pltpu.make_async_copy(
                        p_ref.at[sub(own, k), :], vb, sem_b)
                    cpc = pltpu.make_async_copy(
                        r_ref.at[sub(own, k), :], vc, sem_c)
                    cpa.start()
                    cpb.start()
                    cpc.start()
                    cpa.wait()
                    cpb.wait()
                    cpc.wait()
                    y = va[...] + vb[...] + vc[...]
                    ms = jnp.mean(y * y, axis=-1, keepdims=True)
                    va[...] = y * lax.rsqrt(ms + 1e-6) * g_ref[...]
                    st = pltpu.make_async_copy(
                        va, out_ref.at[sub(own, k), :], sem_out)
                    st.start()
                    st.wait()
                    ag_start(0, k)

        for h in range(1, N_HOP):
            for k in range(SB):
                ag_desc[(0, h - 1, k)].wait_recv()
                ag_desc[(1, h - 1, k)].wait_recv()
                ag_start(h, k)

        for k in range(SB):
            ag_desc[(0, N_HOP - 1, k)].wait_recv()
            ag_desc[(1, N_HOP - 1, k)].wait_recv()
        for dsc in all_desc:
            dsc.wait_send()

    n_sem = 2 * N_HOP * SB
    out, _ = pl.pallas_call(
        body,
        out_shape=(
            jax.ShapeDtypeStruct((M, D), jnp.float32),
            jax.ShapeDtypeStruct((N_HOP, CH, D), jnp.float32),
        ),
        in_specs=[
            pl.BlockSpec(memory_space=pl.ANY),
            pl.BlockSpec(memory_space=pl.ANY),
            pl.BlockSpec(memory_space=pltpu.MemorySpace.VMEM),
        ],
        out_specs=(
            pl.BlockSpec(memory_space=pl.ANY),
            pl.BlockSpec(memory_space=pl.ANY),
        ),
        scratch_shapes=[
            pltpu.VMEM((TR, D), jnp.float32),
            pltpu.VMEM((TR, D), jnp.float32),
            pltpu.VMEM((TR, D), jnp.float32),
            pltpu.SemaphoreType.DMA((n_sem,)),
            pltpu.SemaphoreType.DMA((n_sem,)),
            pltpu.SemaphoreType.DMA((n_sem,)),
            pltpu.SemaphoreType.DMA((n_sem,)),
            pltpu.SemaphoreType.DMA,
            pltpu.SemaphoreType.DMA,
            pltpu.SemaphoreType.DMA,
            pltpu.SemaphoreType.DMA,
            pltpu.SemaphoreType.DMA,
        ],
        compiler_params=pltpu.CompilerParams(
            collective_id=0,
            vmem_limit_bytes=64 * 1024 * 1024,
        ),
    )(partial, resid, gamma)
    return out
